# baseline (speedup 1.0000x reference)
import functools

import jax
import jax.numpy as jnp
from jax.experimental import pallas as pl
from jax.experimental.pallas import tpu as pltpu

_EPS = 1e-6
_RSQ = 0.25
_T0 = 0.25 * (1 + 2.0 ** -23)
_BIG = 1e30
_SENT = 1e9
_B = 128
_UNROLL = 4


def _nms_body(
    n_valid,
    nb,
    xs_ref,
    ys_ref,
    zs_ref,
    mask_ref,
    cnt_ref,
    kx_ref,
    ky_ref,
    kz_ref,
    kn_ref,
):
    b = pl.program_id(0)

    @pl.when(b == 0)
    def _():
        sent = jnp.full((nb, _B), _SENT, jnp.float32)
        kx_ref[:, :] = sent
        ky_ref[:, :] = sent
        kz_ref[:, :] = sent
        kn_ref[0] = 0

    cx = xs_ref[pl.ds(b, 1), :]
    cy = ys_ref[pl.ds(b, 1), :]
    cz = zs_ref[pl.ds(b, 1), :]
    cxt = cx.reshape(_B, 1)
    cyt = cy.reshape(_B, 1)
    czt = cz.reshape(_B, 1)

    def sq_dist(px, py, pz):
        dx = cxt - px + _EPS
        dy = cyt - py + _EPS
        dz = czt - pz + _EPS
        return dx * dx + dy * dy + dz * dz

    kcount = kn_ref[0]

    def prior_body(a4, smin):
        base = a4 * _UNROLL
        x4 = kx_ref[pl.ds(base, _UNROLL), :]
        y4 = ky_ref[pl.ds(base, _UNROLL), :]
        z4 = kz_ref[pl.ds(base, _UNROLL), :]
        for k in range(_UNROLL):
            s = sq_dist(x4[k : k + 1, :], y4[k : k + 1, :], z4[k : k + 1, :])
            smin = jnp.minimum(smin, s)
        return smin

    nprior = (kcount + _B * _UNROLL - 1) // (_B * _UNROLL)
    smin = jax.lax.fori_loop(
        0, nprior, prior_body, jnp.full((_B, _B), _BIG, jnp.float32)
    )
    smin_col = jnp.min(smin, axis=1, keepdims=True)
    presup = jnp.logical_or(smin_col <= _RSQ, smin_col == _T0)

    s_in = sq_dist(cx, cy, cz)
    close_in = jnp.logical_or(s_in <= _RSQ, s_in == _T0)
    ri = jax.lax.broadcasted_iota(jnp.int32, (_B, _B), 0)
    ci = jax.lax.broadcasted_iota(jnp.int32, (_B, _B), 1)
    cl_low = jnp.where(
        jnp.logical_and(close_in, ci < ri), 1.0, 0.0
    )

    sub = jax.lax.broadcasted_iota(jnp.int32, (_B, 1), 0)
    valid = (b * _B + sub) < n_valid
    allowed = jnp.where(
        jnp.logical_and(valid, jnp.logical_not(presup)), 1.0, 0.0
    )

    def fp_cond(carry):
        _, changed = carry
        return changed

    def fp_body(carry):
        k, _ = carry
        hit = jnp.dot(cl_low, k, preferred_element_type=jnp.float32)
        k1 = jnp.where(hit > 0.5, 0.0, allowed)
        hit1 = jnp.dot(cl_low, k1, preferred_element_type=jnp.float32)
        k2 = jnp.where(hit1 > 0.5, 0.0, allowed)
        return k2, jnp.any(k2 != k)

    keep, _ = jax.lax.while_loop(fp_cond, fp_body, (allowed, True))
    mask_ref[pl.ds(b, 1), :] = keep.reshape(1, _B)

    nkept = jnp.sum(keep).astype(jnp.int32)
    row0 = kcount // _B
    off = kcount - row0 * _B

    @pl.when(nkept > 0)
    def _():
        ltri = jnp.where(ri >= ci, 1.0, 0.0)
        rank = jnp.dot(ltri, keep, preferred_element_type=jnp.float32)
        pos = kcount + rank.astype(jnp.int32) - 1
        posm = jnp.bitwise_and(pos, _B - 1)
        smat = jnp.where(
            jnp.logical_and(keep > 0.5, posm == ci), 1.0, 0.0
        )
        hp = jax.lax.Precision.HIGHEST
        vx = jnp.dot(cx, smat, preferred_element_type=jnp.float32, precision=hp)
        vy = jnp.dot(cy, smat, preferred_element_type=jnp.float32, precision=hp)
        vz = jnp.dot(cz, smat, preferred_element_type=jnp.float32, precision=hp)
        li = jax.lax.broadcasted_iota(jnp.int32, (1, _B), 1)
        end = off + nkept
        wr0 = jnp.logical_and(li >= off, li < end)
        wr1 = li < end - _B
        r1 = row0 + 1
        kx_ref[pl.ds(row0, 1), :] = jnp.where(wr0, vx, kx_ref[pl.ds(row0, 1), :])
        ky_ref[pl.ds(row0, 1), :] = jnp.where(wr0, vy, ky_ref[pl.ds(row0, 1), :])
        kz_ref[pl.ds(row0, 1), :] = jnp.where(wr0, vz, kz_ref[pl.ds(row0, 1), :])
        kx_ref[pl.ds(r1, 1), :] = jnp.where(wr1, vx, kx_ref[pl.ds(r1, 1), :])
        ky_ref[pl.ds(r1, 1), :] = jnp.where(wr1, vy, ky_ref[pl.ds(r1, 1), :])
        kz_ref[pl.ds(r1, 1), :] = jnp.where(wr1, vz, kz_ref[pl.ds(r1, 1), :])

    kn_ref[0] = kcount + nkept

    @pl.when(b == nb - 1)
    def _():
        cnt_ref[:, :] = jnp.sum(mask_ref[:, :]).astype(jnp.int32).reshape(1, 1)


def kernel(nodes_dict):
    n = nodes_dict.shape[0]
    nbu = _B * _UNROLL
    npad = ((n + nbu - 1) // nbu) * nbu
    nb = npad // _B
    nodes = jnp.pad(
        nodes_dict, ((0, npad - n), (0, 0)), constant_values=_SENT
    ).astype(jnp.float32)
    xs = nodes[:, 0].reshape(nb, _B)
    ys = nodes[:, 1].reshape(nb, _B)
    zs = nodes[:, 2].reshape(nb, _B)

    mask_f, cnt = pl.pallas_call(
        functools.partial(_nms_body, n, nb),
        grid=(nb,),
        in_specs=[pl.BlockSpec((nb, _B), lambda b: (0, 0))] * 3,
        out_specs=[
            pl.BlockSpec((nb, _B), lambda b: (0, 0)),
            pl.BlockSpec((1, 1), lambda b: (0, 0)),
        ],
        out_shape=[
            jax.ShapeDtypeStruct((nb, _B), jnp.float32),
            jax.ShapeDtypeStruct((1, 1), jnp.int32),
        ],
        scratch_shapes=[
            pltpu.VMEM((nb, _B), jnp.float32),
            pltpu.VMEM((nb, _B), jnp.float32),
            pltpu.VMEM((nb, _B), jnp.float32),
            pltpu.SMEM((1,), jnp.int32),
        ],
    )(xs, ys, zs)

    mask = mask_f.reshape(-1)[:n] > 0.5
    return (mask, cnt.reshape(1))


# speedup vs baseline: 1.1102x; 1.1102x over previous
import functools

import jax
import jax.numpy as jnp
from jax.experimental import pallas as pl
from jax.experimental.pallas import tpu as pltpu

_EPS = 1e-6
_RSQ = 0.25
_T0 = 0.25 * (1 + 2.0 ** -23)
_BIG = 1e30
_SENT = 1e9
_B = 128
_UNROLL = 4


def _nms_body(
    n_valid,
    nb,
    xs_ref,
    ys_ref,
    zs_ref,
    mask_ref,
    cnt_ref,
    kx_ref,
    ky_ref,
    kz_ref,
    kn_ref,
):
    b = pl.program_id(0)

    @pl.when(b == 0)
    def _():
        sent = jnp.full((nb, _B), _SENT, jnp.float32)
        kx_ref[:, :] = sent
        ky_ref[:, :] = sent
        kz_ref[:, :] = sent
        kn_ref[0] = 0

    cx = xs_ref[pl.ds(b, 1), :]
    cy = ys_ref[pl.ds(b, 1), :]
    cz = zs_ref[pl.ds(b, 1), :]
    cxt = cx.reshape(_B, 1)
    cyt = cy.reshape(_B, 1)
    czt = cz.reshape(_B, 1)

    def sq_dist(px, py, pz):
        dx = cxt - px + _EPS
        dy = cyt - py + _EPS
        dz = czt - pz + _EPS
        return dx * dx + dy * dy + dz * dz

    kcount = kn_ref[0]

    def prior_body(a4, smin):
        base = a4 * _UNROLL
        x4 = kx_ref[pl.ds(base, _UNROLL), :]
        y4 = ky_ref[pl.ds(base, _UNROLL), :]
        z4 = kz_ref[pl.ds(base, _UNROLL), :]
        for k in range(_UNROLL):
            s = sq_dist(x4[k : k + 1, :], y4[k : k + 1, :], z4[k : k + 1, :])
            smin = jnp.minimum(smin, s)
        return smin

    nprior = (kcount + _B * _UNROLL - 1) // (_B * _UNROLL)
    smin = jax.lax.fori_loop(
        0, nprior, prior_body, jnp.full((_B, _B), _BIG, jnp.float32)
    )
    smin_col = jnp.min(smin, axis=1, keepdims=True)
    presup = jnp.logical_or(smin_col <= _RSQ, smin_col == _T0)

    s_in = sq_dist(cx, cy, cz)
    close_in = jnp.logical_or(s_in <= _RSQ, s_in == _T0)
    ri = jax.lax.broadcasted_iota(jnp.int32, (_B, _B), 0)
    ci = jax.lax.broadcasted_iota(jnp.int32, (_B, _B), 1)
    cl_low = jnp.where(
        jnp.logical_and(close_in, ci < ri), 1.0, 0.0
    )

    sub = jax.lax.broadcasted_iota(jnp.int32, (_B, 1), 0)
    valid = (b * _B + sub) < n_valid
    allowed = jnp.where(
        jnp.logical_and(valid, jnp.logical_not(presup)), 1.0, 0.0
    )

    def fp_cond(carry):
        _, changed = carry
        return changed

    def fp_body(carry):
        k, _ = carry
        hit = jnp.dot(cl_low, k, preferred_element_type=jnp.float32)
        newk = jnp.where(hit > 0.5, 0.0, allowed)
        return newk, jnp.any(newk != k)

    keep, _ = jax.lax.while_loop(fp_cond, fp_body, (allowed, True))
    mask_ref[pl.ds(b, 1), :] = keep.reshape(1, _B)

    nkept = jnp.sum(keep).astype(jnp.int32)
    row0 = kcount // _B
    off = kcount - row0 * _B

    @pl.when(nkept > 0)
    def _():
        ltri = jnp.where(ri >= ci, 1.0, 0.0)
        rank = jnp.dot(ltri, keep, preferred_element_type=jnp.float32)
        pos = kcount + rank.astype(jnp.int32) - 1
        posm = jnp.bitwise_and(pos, _B - 1)
        smat = jnp.where(
            jnp.logical_and(keep > 0.5, posm == ci), 1.0, 0.0
        )
        hp = jax.lax.Precision.HIGHEST
        vx = jnp.dot(cx, smat, preferred_element_type=jnp.float32, precision=hp)
        vy = jnp.dot(cy, smat, preferred_element_type=jnp.float32, precision=hp)
        vz = jnp.dot(cz, smat, preferred_element_type=jnp.float32, precision=hp)
        li = jax.lax.broadcasted_iota(jnp.int32, (1, _B), 1)
        end = off + nkept
        wr0 = jnp.logical_and(li >= off, li < end)
        wr1 = li < end - _B
        r1 = row0 + 1
        kx_ref[pl.ds(row0, 1), :] = jnp.where(wr0, vx, kx_ref[pl.ds(row0, 1), :])
        ky_ref[pl.ds(row0, 1), :] = jnp.where(wr0, vy, ky_ref[pl.ds(row0, 1), :])
        kz_ref[pl.ds(row0, 1), :] = jnp.where(wr0, vz, kz_ref[pl.ds(row0, 1), :])
        kx_ref[pl.ds(r1, 1), :] = jnp.where(wr1, vx, kx_ref[pl.ds(r1, 1), :])
        ky_ref[pl.ds(r1, 1), :] = jnp.where(wr1, vy, ky_ref[pl.ds(r1, 1), :])
        kz_ref[pl.ds(r1, 1), :] = jnp.where(wr1, vz, kz_ref[pl.ds(r1, 1), :])

    kn_ref[0] = kcount + nkept

    @pl.when(b == nb - 1)
    def _():
        cnt_ref[:, :] = jnp.sum(mask_ref[:, :]).astype(jnp.int32).reshape(1, 1)


def kernel(nodes_dict):
    n = nodes_dict.shape[0]
    nbu = _B * _UNROLL
    npad = ((n + nbu - 1) // nbu) * nbu
    nb = npad // _B
    nodes = jnp.pad(
        nodes_dict, ((0, npad - n), (0, 0)), constant_values=_SENT
    ).astype(jnp.float32)
    xs = nodes[:, 0].reshape(nb, _B)
    ys = nodes[:, 1].reshape(nb, _B)
    zs = nodes[:, 2].reshape(nb, _B)

    mask_f, cnt = pl.pallas_call(
        functools.partial(_nms_body, n, nb),
        grid=(nb,),
        in_specs=[pl.BlockSpec((nb, _B), lambda b: (0, 0))] * 3,
        out_specs=[
            pl.BlockSpec((nb, _B), lambda b: (0, 0)),
            pl.BlockSpec((1, 1), lambda b: (0, 0)),
        ],
        out_shape=[
            jax.ShapeDtypeStruct((nb, _B), jnp.float32),
            jax.ShapeDtypeStruct((1, 1), jnp.int32),
        ],
        scratch_shapes=[
            pltpu.VMEM((nb, _B), jnp.float32),
            pltpu.VMEM((nb, _B), jnp.float32),
            pltpu.VMEM((nb, _B), jnp.float32),
            pltpu.SMEM((1,), jnp.int32),
        ],
    )(xs, ys, zs)

    mask = mask_f.reshape(-1)[:n] > 0.5
    return (mask, cnt.reshape(1))


# speedup vs baseline: 1.1134x; 1.0029x over previous
import functools

import jax
import jax.numpy as jnp
from jax.experimental import pallas as pl
from jax.experimental.pallas import tpu as pltpu

_EPS = 1e-6
_RSQ = 0.25
_T0 = 0.25 * (1 + 2.0 ** -23)
_BIG = 1e30
_SENT = 1e9
_B = 128
_UNROLL = 4


def _nms_body(
    n_valid,
    nb,
    xs_ref,
    ys_ref,
    zs_ref,
    mask_ref,
    cnt_ref,
    kx_ref,
    ky_ref,
    kz_ref,
    kn_ref,
):
    sent = jnp.full((nb, _B), _SENT, jnp.float32)
    kx_ref[:, :] = sent
    ky_ref[:, :] = sent
    kz_ref[:, :] = sent
    kn_ref[0] = 0

    ri = jax.lax.broadcasted_iota(jnp.int32, (_B, _B), 0)
    ci = jax.lax.broadcasted_iota(jnp.int32, (_B, _B), 1)
    ltri = jnp.where(ri >= ci, 1.0, 0.0)

    def block_step(b, carry):
        cx = xs_ref[pl.ds(b, 1), :]
        cy = ys_ref[pl.ds(b, 1), :]
        cz = zs_ref[pl.ds(b, 1), :]
        cxt = cx.reshape(_B, 1)
        cyt = cy.reshape(_B, 1)
        czt = cz.reshape(_B, 1)

        def sq_dist(px, py, pz):
            dx = cxt - px + _EPS
            dy = cyt - py + _EPS
            dz = czt - pz + _EPS
            return dx * dx + dy * dy + dz * dz

        kcount = kn_ref[0]

        def prior_body(a4, smin):
            base = a4 * _UNROLL
            x4 = kx_ref[pl.ds(base, _UNROLL), :]
            y4 = ky_ref[pl.ds(base, _UNROLL), :]
            z4 = kz_ref[pl.ds(base, _UNROLL), :]
            for k in range(_UNROLL):
                s = sq_dist(
                    x4[k : k + 1, :], y4[k : k + 1, :], z4[k : k + 1, :]
                )
                smin = jnp.minimum(smin, s)
            return smin

        nprior = (kcount + _B * _UNROLL - 1) // (_B * _UNROLL)
        smin = jax.lax.fori_loop(
            0, nprior, prior_body, jnp.full((_B, _B), _BIG, jnp.float32)
        )
        smin_col = jnp.min(smin, axis=1, keepdims=True)
        presup = jnp.logical_or(smin_col <= _RSQ, smin_col == _T0)

        s_in = sq_dist(cx, cy, cz)
        close_in = jnp.logical_or(s_in <= _RSQ, s_in == _T0)
        cl_low = jnp.where(
            jnp.logical_and(close_in, ci < ri), 1.0, 0.0
        )

        sub = jax.lax.broadcasted_iota(jnp.int32, (_B, 1), 0)
        valid = (b * _B + sub) < n_valid
        allowed = jnp.where(
            jnp.logical_and(valid, jnp.logical_not(presup)), 1.0, 0.0
        )

        def fp_cond(fp_carry):
            _, changed = fp_carry
            return changed

        def fp_body(fp_carry):
            k, _ = fp_carry
            hit = jnp.dot(cl_low, k, preferred_element_type=jnp.float32)
            newk = jnp.where(hit > 0.5, 0.0, allowed)
            return newk, jnp.any(newk != k)

        keep, _ = jax.lax.while_loop(fp_cond, fp_body, (allowed, True))
        mask_ref[pl.ds(b, 1), :] = keep.reshape(1, _B)

        nkept = jnp.sum(keep).astype(jnp.int32)
        row0 = kcount // _B
        off = kcount - row0 * _B

        @pl.when(nkept > 0)
        def _():
            rank = jnp.dot(ltri, keep, preferred_element_type=jnp.float32)
            pos = kcount + rank.astype(jnp.int32) - 1
            posm = jnp.bitwise_and(pos, _B - 1)
            smat = jnp.where(
                jnp.logical_and(keep > 0.5, posm == ci), 1.0, 0.0
            )
            hp = jax.lax.Precision.HIGHEST
            vx = jnp.dot(
                cx, smat, preferred_element_type=jnp.float32, precision=hp
            )
            vy = jnp.dot(
                cy, smat, preferred_element_type=jnp.float32, precision=hp
            )
            vz = jnp.dot(
                cz, smat, preferred_element_type=jnp.float32, precision=hp
            )
            li = jax.lax.broadcasted_iota(jnp.int32, (1, _B), 1)
            end = off + nkept
            wr0 = jnp.logical_and(li >= off, li < end)
            wr1 = li < end - _B
            r1 = row0 + 1
            kx_ref[pl.ds(row0, 1), :] = jnp.where(
                wr0, vx, kx_ref[pl.ds(row0, 1), :]
            )
            ky_ref[pl.ds(row0, 1), :] = jnp.where(
                wr0, vy, ky_ref[pl.ds(row0, 1), :]
            )
            kz_ref[pl.ds(row0, 1), :] = jnp.where(
                wr0, vz, kz_ref[pl.ds(row0, 1), :]
            )
            kx_ref[pl.ds(r1, 1), :] = jnp.where(
                wr1, vx, kx_ref[pl.ds(r1, 1), :]
            )
            ky_ref[pl.ds(r1, 1), :] = jnp.where(
                wr1, vy, ky_ref[pl.ds(r1, 1), :]
            )
            kz_ref[pl.ds(r1, 1), :] = jnp.where(
                wr1, vz, kz_ref[pl.ds(r1, 1), :]
            )

        kn_ref[0] = kcount + nkept
        return carry

    jax.lax.fori_loop(0, nb, block_step, 0)
    cnt_ref[:, :] = jnp.sum(mask_ref[:, :]).astype(jnp.int32).reshape(1, 1)


def kernel(nodes_dict):
    n = nodes_dict.shape[0]
    nbu = _B * _UNROLL
    npad = ((n + nbu - 1) // nbu) * nbu
    nb = npad // _B
    nodes = jnp.pad(
        nodes_dict, ((0, npad - n), (0, 0)), constant_values=_SENT
    ).astype(jnp.float32)
    xs = nodes[:, 0].reshape(nb, _B)
    ys = nodes[:, 1].reshape(nb, _B)
    zs = nodes[:, 2].reshape(nb, _B)

    mask_f, cnt = pl.pallas_call(
        functools.partial(_nms_body, n, nb),
        grid=(1,),
        in_specs=[pl.BlockSpec((nb, _B), lambda b: (0, 0))] * 3,
        out_specs=[
            pl.BlockSpec((nb, _B), lambda b: (0, 0)),
            pl.BlockSpec((1, 1), lambda b: (0, 0)),
        ],
        out_shape=[
            jax.ShapeDtypeStruct((nb, _B), jnp.float32),
            jax.ShapeDtypeStruct((1, 1), jnp.int32),
        ],
        scratch_shapes=[
            pltpu.VMEM((nb, _B), jnp.float32),
            pltpu.VMEM((nb, _B), jnp.float32),
            pltpu.VMEM((nb, _B), jnp.float32),
            pltpu.SMEM((1,), jnp.int32),
        ],
    )(xs, ys, zs)

    mask = mask_f.reshape(-1)[:n] > 0.5
    return (mask, cnt.reshape(1))


# speedup vs baseline: 1.3336x; 1.1978x over previous
import functools

import jax
import jax.numpy as jnp
from jax.experimental import pallas as pl
from jax.experimental.pallas import tpu as pltpu

_EPS = 1e-6
_RSQ = 0.25
_T0 = 0.25 * (1 + 2.0 ** -23)
_BIG = 1e30
_SENT = 1e9
_B = 128
_BC = 256
_UNROLL = 4


def _nms_body(
    n_valid,
    nbc,
    nbk,
    xs_ref,
    ys_ref,
    zs_ref,
    mask_ref,
    cnt_ref,
    kx_ref,
    ky_ref,
    kz_ref,
    kn_ref,
):
    sent = jnp.full((nbk, _B), _SENT, jnp.float32)
    kx_ref[:, :] = sent
    ky_ref[:, :] = sent
    kz_ref[:, :] = sent
    kn_ref[0] = 0

    ri = jax.lax.broadcasted_iota(jnp.int32, (_BC, _BC), 0)
    ci = jax.lax.broadcasted_iota(jnp.int32, (_BC, _BC), 1)
    ltri = jnp.where(ri >= ci, 1.0, 0.0)
    ci128 = jax.lax.broadcasted_iota(jnp.int32, (_B, _B), 1)
    li128 = jax.lax.broadcasted_iota(jnp.int32, (1, _B), 1)

    def block_step(b, carry):
        cx = xs_ref[pl.ds(b, 1), :]
        cy = ys_ref[pl.ds(b, 1), :]
        cz = zs_ref[pl.ds(b, 1), :]
        cxt = cx.reshape(_BC, 1)
        cyt = cy.reshape(_BC, 1)
        czt = cz.reshape(_BC, 1)

        def sq_dist(px, py, pz):
            dx = cxt - px + _EPS
            dy = cyt - py + _EPS
            dz = czt - pz + _EPS
            return dx * dx + dy * dy + dz * dz

        kcount = kn_ref[0]

        def prior_body(a4, smin):
            base = a4 * _UNROLL
            x4 = kx_ref[pl.ds(base, _UNROLL), :]
            y4 = ky_ref[pl.ds(base, _UNROLL), :]
            z4 = kz_ref[pl.ds(base, _UNROLL), :]
            for k in range(_UNROLL):
                s = sq_dist(
                    x4[k : k + 1, :], y4[k : k + 1, :], z4[k : k + 1, :]
                )
                smin = jnp.minimum(smin, s)
            return smin

        nprior = (kcount + _B * _UNROLL - 1) // (_B * _UNROLL)
        smin = jax.lax.fori_loop(
            0, nprior, prior_body, jnp.full((_BC, _B), _BIG, jnp.float32)
        )
        smin_col = jnp.min(smin, axis=1, keepdims=True)
        presup = jnp.logical_or(smin_col <= _RSQ, smin_col == _T0)

        s_in = sq_dist(cx, cy, cz)
        close_in = jnp.logical_or(s_in <= _RSQ, s_in == _T0)
        cl_low = jnp.where(
            jnp.logical_and(close_in, ci < ri), 1.0, 0.0
        )

        sub = jax.lax.broadcasted_iota(jnp.int32, (_BC, 1), 0)
        valid = (b * _BC + sub) < n_valid
        allowed = jnp.where(
            jnp.logical_and(valid, jnp.logical_not(presup)), 1.0, 0.0
        )

        def fp_cond(fp_carry):
            _, changed = fp_carry
            return changed

        def fp_body(fp_carry):
            k, _ = fp_carry
            hit = jnp.dot(cl_low, k, preferred_element_type=jnp.float32)
            newk = jnp.where(hit > 0.5, 0.0, allowed)
            return newk, jnp.any(newk != k)

        keep, _ = jax.lax.while_loop(fp_cond, fp_body, (allowed, True))
        mask_ref[pl.ds(b, 1), :] = keep.reshape(1, _BC)

        rank = jnp.dot(ltri, keep, preferred_element_type=jnp.float32)
        nkept = jnp.sum(keep).astype(jnp.int32)

        def scatter_half(h, start, ch, cxh, cyh, czh, keep_h, rank_h):
            @pl.when(ch > 0)
            def _():
                pos = kcount + rank_h.astype(jnp.int32) - 1
                posm = jnp.bitwise_and(pos, _B - 1)
                smat = jnp.where(
                    jnp.logical_and(keep_h > 0.5, posm == ci128), 1.0, 0.0
                )
                hp = jax.lax.Precision.HIGHEST
                vx = jnp.dot(
                    cxh, smat, preferred_element_type=jnp.float32, precision=hp
                )
                vy = jnp.dot(
                    cyh, smat, preferred_element_type=jnp.float32, precision=hp
                )
                vz = jnp.dot(
                    czh, smat, preferred_element_type=jnp.float32, precision=hp
                )
                row0 = start // _B
                off = start - row0 * _B
                end = off + ch
                wr0 = jnp.logical_and(li128 >= off, li128 < end)
                wr1 = li128 < end - _B
                r1 = row0 + 1
                kx_ref[pl.ds(row0, 1), :] = jnp.where(
                    wr0, vx, kx_ref[pl.ds(row0, 1), :]
                )
                ky_ref[pl.ds(row0, 1), :] = jnp.where(
                    wr0, vy, ky_ref[pl.ds(row0, 1), :]
                )
                kz_ref[pl.ds(row0, 1), :] = jnp.where(
                    wr0, vz, kz_ref[pl.ds(row0, 1), :]
                )
                kx_ref[pl.ds(r1, 1), :] = jnp.where(
                    wr1, vx, kx_ref[pl.ds(r1, 1), :]
                )
                ky_ref[pl.ds(r1, 1), :] = jnp.where(
                    wr1, vy, ky_ref[pl.ds(r1, 1), :]
                )
                kz_ref[pl.ds(r1, 1), :] = jnp.where(
                    wr1, vz, kz_ref[pl.ds(r1, 1), :]
                )

        keep0 = keep[: _B, :]
        keep1 = keep[_B :, :]
        c0 = jnp.sum(keep0).astype(jnp.int32)
        c1 = nkept - c0
        scatter_half(
            0, kcount, c0, cx[:, : _B], cy[:, : _B], cz[:, : _B],
            keep0, rank[: _B, :],
        )
        scatter_half(
            1, kcount + c0, c1, cx[:, _B :], cy[:, _B :], cz[:, _B :],
            keep1, rank[_B :, :],
        )
        kn_ref[0] = kcount + nkept
        return carry

    jax.lax.fori_loop(0, nbc, block_step, 0)
    cnt_ref[:, :] = jnp.sum(mask_ref[:, :]).astype(jnp.int32).reshape(1, 1)


def kernel(nodes_dict):
    n = nodes_dict.shape[0]
    nbu = _B * _UNROLL * 2
    npad = ((n + nbu - 1) // nbu) * nbu
    nbc = npad // _BC
    nbk = npad // _B
    nodes = jnp.pad(
        nodes_dict, ((0, npad - n), (0, 0)), constant_values=_SENT
    ).astype(jnp.float32)
    xs = nodes[:, 0].reshape(nbc, _BC)
    ys = nodes[:, 1].reshape(nbc, _BC)
    zs = nodes[:, 2].reshape(nbc, _BC)

    mask_f, cnt = pl.pallas_call(
        functools.partial(_nms_body, n, nbc, nbk),
        grid=(1,),
        in_specs=[pl.BlockSpec((nbc, _BC), lambda b: (0, 0))] * 3,
        out_specs=[
            pl.BlockSpec((nbc, _BC), lambda b: (0, 0)),
            pl.BlockSpec((1, 1), lambda b: (0, 0)),
        ],
        out_shape=[
            jax.ShapeDtypeStruct((nbc, _BC), jnp.float32),
            jax.ShapeDtypeStruct((1, 1), jnp.int32),
        ],
        scratch_shapes=[
            pltpu.VMEM((nbk, _B), jnp.float32),
            pltpu.VMEM((nbk, _B), jnp.float32),
            pltpu.VMEM((nbk, _B), jnp.float32),
            pltpu.SMEM((1,), jnp.int32),
        ],
    )(xs, ys, zs)

    mask = mask_f.reshape(-1)[:n] > 0.5
    return (mask, cnt.reshape(1))
